# Initial kernel scaffold; baseline (speedup 1.0000x reference)
import jax
import jax.numpy as jnp
from jax.experimental import pallas as pl


def kernel(V, E, edge_index, rev_edge_index, Wi_atom, Wi_bond, Wh_atom, Wh_bond, Wo_atom, Wf_atom, Wf_bond):
    raise NotImplementedError("write your pallas kernel here")


# speedup vs baseline: 2.4983x; 2.4983x over previous
import jax
import jax.numpy as jnp
from jax import lax
from jax.experimental import pallas as pl
from jax.experimental.pallas import tpu as pltpu
from jax.experimental.pallas import tpu_sc as plsc

N = 10000
M = 320000
D = 128
D_E = 16
CH = 80
NC = 2
NS = 16
NW = NC * NS
ROWS = M // CH
RPW = ROWS // NW

_mesh = plsc.VectorSubcoreMesh(core_axis_name="c", subcore_axis_name="s")


def _sc_kernel(do_scatter, do_vgather, do_hrev):
    scratch = [
        pltpu.VMEM_SHARED((N, D), jnp.float32),
        pltpu.VMEM((RPW, CH), jnp.int32),
        pltpu.VMEM((CH, D), jnp.float32),
        pltpu.VMEM((CH, D), jnp.float32),
        pltpu.SemaphoreType.DMA,
    ]

    def body(*refs):
        it = iter(refs)
        hb = next(it) if (do_scatter or do_hrev) else None
        table = next(it) if (do_scatter or do_vgather) else None
        widx_h = next(it) if do_scatter else None
        vidx_h = next(it) if do_vgather else None
        ridx_h = next(it) if do_hrev else None
        part_o = next(it) if do_scatter else None
        vout_o = next(it) if do_vgather else None
        hrev_o = next(it) if do_hrev else None
        acc, idx, vals, rows, sem = it

        c = lax.axis_index("c")
        s = lax.axis_index("s")
        wid = s * NC + c
        base = wid * RPW

        @pl.when(s == 0)
        def _():
            pltpu.sync_copy(table, acc)
        plsc.subcore_barrier()

        if do_scatter:
            pltpu.sync_copy(widx_h.at[wid], idx)

            def sbody(j, carry):
                pltpu.sync_copy(hb.at[pl.ds((base + j) * CH, CH)], vals)
                pltpu.sync_copy(vals, acc.at[idx.at[j]], add=True)
                return carry

            lax.fori_loop(0, RPW, sbody, 0)
            plsc.subcore_barrier()

            @pl.when(s == 0)
            def _():
                pltpu.sync_copy(acc, part_o.at[c])

        if do_vgather:
            pltpu.sync_copy(vidx_h.at[wid], idx)

            def gbody(j, carry):
                pltpu.async_copy(acc.at[idx.at[j]], vals, sem).wait()
                pltpu.sync_copy(vals, vout_o.at[pl.ds((base + j) * CH, CH)])
                return carry

            lax.fori_loop(0, RPW, gbody, 0)

        if do_hrev:
            pltpu.sync_copy(ridx_h.at[wid], idx)

            def rbody(j, carry):
                pltpu.async_copy(hb.at[idx.at[j]], rows, sem).wait()
                pltpu.sync_copy(rows, hrev_o.at[pl.ds((base + j) * CH, CH)])
                return carry

            lax.fori_loop(0, RPW, rbody, 0)

    outs = []
    if do_scatter:
        outs.append(jax.ShapeDtypeStruct((NC, N, D), jnp.float32))
    if do_vgather:
        outs.append(jax.ShapeDtypeStruct((M, D), jnp.float32))
    if do_hrev:
        outs.append(jax.ShapeDtypeStruct((M, D), jnp.float32))

    return pl.kernel(body, out_type=tuple(outs), mesh=_mesh,
                     scratch_types=scratch)


_sc_vgather = _sc_kernel(do_scatter=False, do_vgather=True, do_hrev=False)
_sc_scatter = _sc_kernel(do_scatter=True, do_vgather=False, do_hrev=False)
_sc_gather2 = _sc_kernel(do_scatter=False, do_vgather=True, do_hrev=True)


BM = 2000
BN = 2000


def _k1_body(vv, e, w1, w2, out):
    out[...] = jnp.maximum(vv[...] @ w1[...] + e[...] @ w2[...], 0.0)


def _k1(Vv, E, W1, W2):
    return pl.pallas_call(
        _k1_body,
        grid=(M // BM,),
        in_specs=[
            pl.BlockSpec((BM, D), lambda i: (i, 0)),
            pl.BlockSpec((BM, D_E), lambda i: (i, 0)),
            pl.BlockSpec((D, D), lambda i: (0, 0)),
            pl.BlockSpec((D_E, D), lambda i: (0, 0)),
        ],
        out_specs=pl.BlockSpec((BM, D), lambda i: (i, 0)),
        out_shape=jax.ShapeDtypeStruct((M, D), jnp.float32),
    )(Vv, E, W1, W2)


def _kadd_body(p, out):
    out[...] = p[0] + p[1]


def _kadd(P):
    return pl.pallas_call(
        _kadd_body,
        grid=(N // BN,),
        in_specs=[pl.BlockSpec((NC, BN, D), lambda i: (0, i, 0))],
        out_specs=pl.BlockSpec((BN, D), lambda i: (i, 0)),
        out_shape=jax.ShapeDtypeStruct((N, D), jnp.float32),
    )(P)


def _k3_body(hb, av, hrev, wh, wf1, wf2, out):
    m = (av[...] - hrev[...]) @ wh[...]
    z = hb[...] @ wf1[...] + m @ wf2[...]
    out[...] = hb[...] + jnp.maximum(z, 0.0)


def _k3(Hb, Av, Hrev, Wh, Wf1, Wf2):
    return pl.pallas_call(
        _k3_body,
        grid=(M // BM,),
        in_specs=[
            pl.BlockSpec((BM, D), lambda i: (i, 0)),
            pl.BlockSpec((BM, D), lambda i: (i, 0)),
            pl.BlockSpec((BM, D), lambda i: (i, 0)),
            pl.BlockSpec((D, D), lambda i: (0, 0)),
            pl.BlockSpec((D, D), lambda i: (0, 0)),
            pl.BlockSpec((D, D), lambda i: (0, 0)),
        ],
        out_specs=pl.BlockSpec((BM, D), lambda i: (i, 0)),
        out_shape=jax.ShapeDtypeStruct((M, D), jnp.float32),
    )(Hb, Av, Hrev, Wh, Wf1, Wf2)


def _k4_body(v, p, wo1, wo2, out):
    asum = p[0] + p[1]
    out[...] = jnp.maximum(v[...] @ wo1[...] + asum @ wo2[...], 0.0)


def _k4(V, P, Wo1, Wo2):
    return pl.pallas_call(
        _k4_body,
        grid=(N // BN,),
        in_specs=[
            pl.BlockSpec((BN, D), lambda i: (i, 0)),
            pl.BlockSpec((NC, BN, D), lambda i: (0, i, 0)),
            pl.BlockSpec((D, D), lambda i: (0, 0)),
            pl.BlockSpec((D, D), lambda i: (0, 0)),
        ],
        out_specs=pl.BlockSpec((BN, D), lambda i: (i, 0)),
        out_shape=jax.ShapeDtypeStruct((N, D), jnp.float32),
    )(V, P, Wo1, Wo2)


def kernel(V, E, edge_index, rev_edge_index, Wi_atom, Wi_bond,
           Wh_atom, Wh_bond, Wo_atom, Wf_atom, Wf_bond):
    v3d = edge_index[0].reshape(NW, RPW, CH)
    w3d = edge_index[1].reshape(NW, RPW, CH)
    rev3d = rev_edge_index.reshape(NW, RPW, CH)
    zeros = jnp.zeros((N, D), jnp.float32)

    (Vv,) = _sc_vgather(V, v3d)
    Hb = _k1(Vv, E, Wi_bond[:D], Wi_bond[D:])

    for _ in range(2):
        (P,) = _sc_scatter(Hb, zeros, w3d)
        Asum = _kadd(P)
        Av, Hrev = _sc_gather2(Hb, Asum, v3d, rev3d)
        Hb = _k3(Hb, Av, Hrev, Wh_bond, Wf_bond[:D], Wf_bond[D:])

    (P,) = _sc_scatter(Hb, zeros, w3d)
    return _k4(V, P, Wo_atom[:D], Wo_atom[D:])
